# baseline (speedup 1.0000x reference)
import jax
import jax.numpy as jnp
from jax import lax
from jax.experimental import pallas as pl
from jax.experimental.pallas import tpu as pltpu
from jax.experimental.pallas import tpu_sc as plsc

N_NODES = 10000
D_FEAT = 128
N_EDGES = 320000
EPS = 1e-6

NC = 2
NS = 16
L = 16
NW = NC * NS
E_W = N_EDGES // NW
CHUNK = 80
NCHUNK = E_W // CHUNK
NPAIR = (NCHUNK - 1) // 2
NGROUP = CHUNK // L
NSLICE = D_FEAT // L


def _sqrt_newton(x):
    i = plsc.bitcast(x, jnp.int32)
    i = jnp.int32(0x5F3759DF) - lax.shift_right_arithmetic(i, 1)
    y = plsc.bitcast(i, jnp.float32)
    half = x * 0.5
    for _ in range(3):
        y = y * (1.5 - half * y * y)
    return x * y


def _body(src_hbm, dst_hbm, z_hbm, out_hbm,
          idx_s, idx_d, rows_s0, rows_d0, rows_s1, rows_d1, tbuf, out_v,
          sem_s0, sem_d0, sem_s1, sem_d1):
    wid = lax.axis_index("s") * NC + lax.axis_index("c")
    base = wid * E_W

    pltpu.sync_copy(src_hbm.at[pl.ds(base, E_W)], idx_s)
    pltpu.sync_copy(dst_hbm.at[pl.ds(base, E_W)], idx_d)

    bufs = ((rows_s0, rows_d0, sem_s0, sem_d0),
            (rows_s1, rows_d1, sem_s1, sem_d1))

    def issue(j, buf):
        rs, rd, ss, sd = buf
        off = j * CHUNK
        pltpu.async_copy(z_hbm.at[idx_s.at[pl.ds(off, CHUNK)]], rs, ss)
        pltpu.async_copy(z_hbm.at[idx_d.at[pl.ds(off, CHUNK)]], rd, sd)

    def wait(j, buf):
        rs, rd, ss, sd = buf
        off = j * CHUNK
        pltpu.make_async_copy(z_hbm.at[idx_s.at[pl.ds(off, CHUNK)]], rs, ss).wait()
        pltpu.make_async_copy(z_hbm.at[idx_d.at[pl.ds(off, CHUNK)]], rd, sd).wait()

    lane = lax.iota(jnp.int32, L)
    colbase = lane * L

    def compute(j, buf):
        rs, rd, _, _ = buf
        off = j * CHUNK
        for g in range(NGROUP):
            @plsc.parallel_loop(0, L, step=1, unroll=4)
            def edge_body(el):
                e = g * L + el
                acc = jnp.zeros((L,), jnp.float32)
                for s in range(NSLICE):
                    a = rs[e, pl.ds(s * L, L)]
                    b = rd[e, pl.ds(s * L, L)]
                    d = (a - b) + EPS
                    acc = acc + d * d
                tbuf[pl.ds(el * L, L)] = acc
            tot = jnp.zeros((L,), jnp.float32)
            for c in range(L):
                tot = tot + plsc.load_gather(tbuf, [colbase + c])
            out_v[pl.ds(off + g * L, L)] = _sqrt_newton(tot)

    issue(0, bufs[0])

    def pair_body(p, carry):
        j0 = 2 * p
        issue(j0 + 1, bufs[1])
        wait(j0, bufs[0])
        compute(j0, bufs[0])
        issue(j0 + 2, bufs[0])
        wait(j0 + 1, bufs[1])
        compute(j0 + 1, bufs[1])
        return carry

    lax.fori_loop(0, NPAIR, pair_body, 0)
    wait(NCHUNK - 1, bufs[0])
    compute(NCHUNK - 1, bufs[0])

    pltpu.sync_copy(out_v, out_hbm.at[pl.ds(base, E_W)])


def _scratch_types():
    return (
        [pltpu.VMEM((E_W,), jnp.int32)] * 2
        + [pltpu.VMEM((CHUNK, D_FEAT), jnp.float32)] * 4
        + [pltpu.VMEM((L * L,), jnp.float32), pltpu.VMEM((E_W,), jnp.float32)]
        + [pltpu.SemaphoreType.DMA] * 4
    )


@jax.jit
def kernel(z, edge_index):
    src = edge_index[0].astype(jnp.int32)
    dst = edge_index[1].astype(jnp.int32)
    mesh = plsc.VectorSubcoreMesh(core_axis_name="c", subcore_axis_name="s",
                                  num_cores=NC, num_subcores=NS)
    f = pl.kernel(
        _body,
        out_type=jax.ShapeDtypeStruct((N_EDGES,), jnp.float32),
        mesh=mesh,
        compiler_params=pltpu.CompilerParams(needs_layout_passes=False),
        scratch_types=_scratch_types(),
    )
    return f(src, dst, z)


# speedup vs baseline: 1.3115x; 1.3115x over previous
import jax
import jax.numpy as jnp
from jax import lax
from jax.experimental import pallas as pl
from jax.experimental.pallas import tpu as pltpu
from jax.experimental.pallas import tpu_sc as plsc

N_NODES = 10000
D_FEAT = 128
N_EDGES = 320000
EPS = 1e-6

NC = 2
NS = 16
L = 16
NW = NC * NS
E_W = N_EDGES // NW
CHUNK = 80
NCHUNK = E_W // CHUNK
NPAIR = (NCHUNK - 1) // 2
NGROUP = CHUNK // L
NSLICE = D_FEAT // L


def _sqrt_newton(x):
    i = plsc.bitcast(x, jnp.int32)
    i = jnp.int32(0x5F3759DF) - lax.shift_right_arithmetic(i, 1)
    y = plsc.bitcast(i, jnp.float32)
    half = x * 0.5
    for _ in range(2):
        y = y * (1.5 - half * y * y)
    return x * y


def _body(src_hbm, dst_hbm, z_hbm, out_hbm,
          idx_s, idx_d, rows_s0, rows_d0, rows_s1, rows_d1, tbuf, out_v,
          sem_s0, sem_d0, sem_s1, sem_d1):
    wid = lax.axis_index("s") * NC + lax.axis_index("c")
    base = wid * E_W

    pltpu.sync_copy(src_hbm.at[pl.ds(base, E_W)], idx_s)
    pltpu.sync_copy(dst_hbm.at[pl.ds(base, E_W)], idx_d)

    bufs = ((rows_s0, rows_d0, sem_s0, sem_d0),
            (rows_s1, rows_d1, sem_s1, sem_d1))

    def issue(j, buf):
        rs, rd, ss, sd = buf
        off = j * CHUNK
        pltpu.async_copy(z_hbm.at[idx_s.at[pl.ds(off, CHUNK)]], rs, ss)
        pltpu.async_copy(z_hbm.at[idx_d.at[pl.ds(off, CHUNK)]], rd, sd)

    def wait(j, buf):
        rs, rd, ss, sd = buf
        off = j * CHUNK
        pltpu.make_async_copy(z_hbm.at[idx_s.at[pl.ds(off, CHUNK)]], rs, ss).wait()
        pltpu.make_async_copy(z_hbm.at[idx_d.at[pl.ds(off, CHUNK)]], rd, sd).wait()

    lane = lax.iota(jnp.int32, L)
    colbase = lane * L

    def compute(j, buf):
        rs, rd, _, _ = buf
        off = j * CHUNK
        for g in range(NGROUP):
            @plsc.parallel_loop(0, L, step=1, unroll=2)
            def edge_body(el):
                e = g * L + el
                acc = jnp.zeros((L,), jnp.float32)
                for s in range(NSLICE):
                    a = rs[e, pl.ds(s * L, L)]
                    b = rd[e, pl.ds(s * L, L)]
                    d = (a - b) + EPS
                    acc = acc + d * d
                tbuf[pl.ds(el * L, L)] = acc
            tots = [plsc.load_gather(tbuf, [colbase + c]) for c in range(4)]
            for c in range(4, L):
                tots[c % 4] = tots[c % 4] + plsc.load_gather(tbuf, [colbase + c])
            tot = (tots[0] + tots[1]) + (tots[2] + tots[3])
            out_v[pl.ds(off + g * L, L)] = _sqrt_newton(tot)

    issue(0, bufs[0])

    def pair_body(p, carry):
        j0 = 2 * p
        issue(j0 + 1, bufs[1])
        wait(j0, bufs[0])
        compute(j0, bufs[0])
        issue(j0 + 2, bufs[0])
        wait(j0 + 1, bufs[1])
        compute(j0 + 1, bufs[1])
        return carry

    lax.fori_loop(0, NPAIR, pair_body, 0)
    wait(NCHUNK - 1, bufs[0])
    compute(NCHUNK - 1, bufs[0])

    pltpu.sync_copy(out_v, out_hbm.at[pl.ds(base, E_W)])


def _scratch_types():
    return (
        [pltpu.VMEM((E_W,), jnp.int32)] * 2
        + [pltpu.VMEM((CHUNK, D_FEAT), jnp.float32)] * 4
        + [pltpu.VMEM((L * L,), jnp.float32), pltpu.VMEM((E_W,), jnp.float32)]
        + [pltpu.SemaphoreType.DMA] * 4
    )


@jax.jit
def kernel(z, edge_index):
    src = edge_index[0].astype(jnp.int32)
    dst = edge_index[1].astype(jnp.int32)
    mesh = plsc.VectorSubcoreMesh(core_axis_name="c", subcore_axis_name="s",
                                  num_cores=NC, num_subcores=NS)
    f = pl.kernel(
        _body,
        out_type=jax.ShapeDtypeStruct((N_EDGES,), jnp.float32),
        mesh=mesh,
        compiler_params=pltpu.CompilerParams(needs_layout_passes=False),
        scratch_types=_scratch_types(),
    )
    return f(src, dst, z)
